# baseline (speedup 1.0000x reference)
import functools

import jax
import jax.numpy as jnp
from jax import lax
from jax.experimental import pallas as pl
from jax.experimental.pallas import tpu as pltpu
from jax.experimental.pallas import tpu_sc as plsc

NC = 2
NS = 16
NW = NC * NS


def _sc_gather_scatter_add(n_rows, row_w, n_chunks, chunk, nbuf):
    rps = n_rows // NS
    mesh = plsc.VectorSubcoreMesh(core_axis_name="core",
                                  subcore_axis_name="subcore")

    @functools.partial(
        pl.kernel,
        out_type=jax.ShapeDtypeStruct((NC, n_rows, row_w), jnp.float32),
        mesh=mesh,
        compiler_params=pltpu.CompilerParams(use_tc_tiling_on_sc=False),
        scratch_types=[
            pltpu.VMEM((n_chunks, chunk), jnp.int32),
            pltpu.VMEM((n_chunks, chunk), jnp.int32),
            pltpu.VMEM((nbuf * chunk, row_w), jnp.float32),
            pltpu.VMEM_SHARED((n_rows, row_w), jnp.float32),
            [pltpu.SemaphoreType.DMA for _ in range(nbuf)],
        ],
    )
    def body(table_hbm, idx_hbm, dst_hbm, zero_hbm, out_hbm,
             idx_v, dst_v, buf_all, acc_sh, sems):
        bufs = [buf_all.at[pl.ds(b * chunk, chunk)] for b in range(nbuf)]
        cid = lax.axis_index("core")
        sid = lax.axis_index("subcore")
        wid = sid * NC + cid
        pltpu.sync_copy(zero_hbm.at[pl.ds(sid * rps, rps)],
                        acc_sh.at[pl.ds(sid * rps, rps)])
        pltpu.sync_copy(idx_hbm.at[wid], idx_v)
        pltpu.sync_copy(dst_hbm.at[wid], dst_v)
        plsc.subcore_barrier()

        @pl.loop(0, n_chunks, step=nbuf)
        def _(i):
            for b in range(nbuf):
                pltpu.async_copy(table_hbm.at[idx_v.at[i + b]],
                                 bufs[b], sems[b])
            for b in range(nbuf):
                pltpu.make_async_copy(table_hbm.at[idx_v.at[i + b]],
                                      bufs[b], sems[b]).wait()
                pltpu.sync_copy(bufs[b], acc_sh.at[dst_v.at[i + b]],
                                add=True)

        plsc.subcore_barrier()
        pltpu.sync_copy(acc_sh.at[pl.ds(sid * rps, rps)],
                        out_hbm.at[cid].at[pl.ds(sid * rps, rps)])

    return body


def _dense_stage(n_nodes, hidden, n_rel, n_cls, bn):
    def body(hp_ref, root1_ref, b1_ref, w2t_ref, root2_ref, b2_ref,
             p_ref, oroot_ref):
        h = hp_ref[0] + hp_ref[1] + root1_ref[...] + b1_ref[...]
        h = jnp.maximum(h, 0.0)
        oroot_ref[...] = (
            jnp.dot(h, root2_ref[...], preferred_element_type=jnp.float32)
            + b2_ref[...])
        p_ref[...] = jnp.dot(h, w2t_ref[...],
                             preferred_element_type=jnp.float32)

    return pl.pallas_call(
        body,
        grid=(n_nodes // bn,),
        in_specs=[
            pl.BlockSpec((2, bn, hidden), lambda i: (0, i, 0)),
            pl.BlockSpec((bn, hidden), lambda i: (i, 0)),
            pl.BlockSpec((1, hidden), lambda i: (0, 0)),
            pl.BlockSpec((hidden, n_rel * n_cls), lambda i: (0, 0)),
            pl.BlockSpec((hidden, n_cls), lambda i: (0, 0)),
            pl.BlockSpec((1, n_cls), lambda i: (0, 0)),
        ],
        out_specs=[
            pl.BlockSpec((bn, n_rel * n_cls), lambda i: (i, 0)),
            pl.BlockSpec((bn, n_cls), lambda i: (i, 0)),
        ],
        out_shape=[
            jax.ShapeDtypeStruct((n_nodes, n_rel * n_cls), jnp.float32),
            jax.ShapeDtypeStruct((n_nodes, n_cls), jnp.float32),
        ],
    )


def _final_stage(n_nodes, n_cls, bn):
    def body(o2_ref, oroot_ref, out_ref):
        x = o2_ref[0] + o2_ref[1] + oroot_ref[...]
        m = jnp.max(x, axis=1, keepdims=True)
        e = jnp.exp(x - m)
        lse = jnp.log(jnp.sum(e, axis=1, keepdims=True))
        out_ref[...] = x - m - lse

    return pl.pallas_call(
        body,
        grid=(n_nodes // bn,),
        in_specs=[
            pl.BlockSpec((2, bn, n_cls), lambda i: (0, i, 0)),
            pl.BlockSpec((bn, n_cls), lambda i: (i, 0)),
        ],
        out_specs=pl.BlockSpec((bn, n_cls), lambda i: (i, 0)),
        out_shape=jax.ShapeDtypeStruct((n_nodes, n_cls), jnp.float32),
    )


def kernel(edge_index, edge_type, W1, root1, b1, W2, root2, b2):
    n_rel, n_nodes, hidden = W1.shape
    n_cls = root2.shape[1]
    n_edges = edge_type.shape[0]

    chunk1, nbuf1 = 80, 1
    chunk2, nbuf2 = 80, 25
    epw = n_edges // NW

    src = edge_index[0]
    dst = edge_index[1]
    n_pad = ((n_nodes + 8 * NS - 1) // (8 * NS)) * (8 * NS)

    def _shard(x, fill, chunk, nbuf):
        n_chunks = -(-(-(-epw // chunk)) // nbuf) * nbuf
        pad = n_chunks * chunk - epw
        if fill is None:
            fill_vec = n_nodes + jnp.arange(pad, dtype=jnp.int32) % (
                n_pad - n_nodes)
            pad_blk = jnp.broadcast_to(fill_vec, (NW, pad))
            x = jnp.concatenate([x.reshape(NW, epw), pad_blk], axis=1)
        else:
            x = jnp.pad(x.reshape(NW, epw), ((0, 0), (0, pad)),
                        constant_values=fill)
        return x.reshape(NW, n_chunks, chunk), n_chunks

    idx1, nch1 = _shard(edge_type * n_nodes + src, 0, chunk1, nbuf1)
    idx2, nch2 = _shard(src * n_rel + edge_type, 0, chunk2, nbuf2)
    dstr1, _ = _shard(dst, None, chunk1, nbuf1)
    dstr2, _ = _shard(dst, None, chunk2, nbuf2)
    zeros_h = jnp.zeros((n_pad, hidden), jnp.float32)
    zeros_c = jnp.zeros((n_pad, n_cls), jnp.float32)

    l1 = _sc_gather_scatter_add(n_pad, hidden, nch1, chunk1, nbuf1)
    hp = l1(W1.reshape(n_rel * n_nodes, hidden), idx1, dstr1, zeros_h)

    w2t = W2.transpose(1, 0, 2).reshape(hidden, n_rel * n_cls)
    h = jnp.maximum(hp[0, :n_nodes] + hp[1, :n_nodes] + root1 + b1, 0.0)
    oroot = h @ root2 + b2
    p = h @ w2t

    l2 = _sc_gather_scatter_add(n_pad, n_cls, nch2, chunk2, nbuf2)
    out2 = l2(p.reshape(n_nodes * n_rel, n_cls), idx2, dstr2, zeros_c)

    x = out2[0, :n_nodes] + out2[1, :n_nodes] + oroot
    return jax.nn.log_softmax(x, axis=1)


# speedup vs baseline: 1.0198x; 1.0198x over previous
import functools

import jax
import jax.numpy as jnp
from jax import lax
from jax.experimental import pallas as pl
from jax.experimental.pallas import tpu as pltpu
from jax.experimental.pallas import tpu_sc as plsc

NC = 2
NS = 16
NW = NC * NS


def _sc_gather_scatter_add(n_rows, row_w, n_chunks, chunk, nbuf):
    rps = n_rows // NS
    mesh = plsc.VectorSubcoreMesh(core_axis_name="core",
                                  subcore_axis_name="subcore")

    @functools.partial(
        pl.kernel,
        out_type=jax.ShapeDtypeStruct((NC, n_rows, row_w), jnp.float32),
        mesh=mesh,
        compiler_params=pltpu.CompilerParams(use_tc_tiling_on_sc=False),
        scratch_types=[
            pltpu.VMEM((n_chunks, chunk), jnp.int32),
            pltpu.VMEM((n_chunks, chunk), jnp.int32),
            pltpu.VMEM((nbuf * chunk, row_w), jnp.float32),
            pltpu.VMEM_SHARED((n_rows, row_w), jnp.float32),
            [pltpu.SemaphoreType.DMA for _ in range(nbuf)],
        ],
    )
    def body(table_hbm, idx_hbm, dst_hbm, zero_hbm, out_hbm,
             idx_v, dst_v, buf_all, acc_sh, sems):
        bufs = [buf_all.at[pl.ds(b * chunk, chunk)] for b in range(nbuf)]
        cid = lax.axis_index("core")
        sid = lax.axis_index("subcore")
        wid = sid * NC + cid
        pltpu.sync_copy(zero_hbm.at[pl.ds(sid * rps, rps)],
                        acc_sh.at[pl.ds(sid * rps, rps)])
        pltpu.sync_copy(idx_hbm.at[wid], idx_v)
        pltpu.sync_copy(dst_hbm.at[wid], dst_v)
        plsc.subcore_barrier()

        @pl.loop(0, n_chunks, step=nbuf)
        def _(i):
            for b in range(nbuf):
                pltpu.async_copy(table_hbm.at[idx_v.at[i + b]],
                                 bufs[b], sems[b])
            for b in range(nbuf):
                pltpu.make_async_copy(table_hbm.at[idx_v.at[i + b]],
                                      bufs[b], sems[b]).wait()
                pltpu.sync_copy(bufs[b], acc_sh.at[dst_v.at[i + b]],
                                add=True)

        plsc.subcore_barrier()
        pltpu.sync_copy(acc_sh.at[pl.ds(sid * rps, rps)],
                        out_hbm.at[cid].at[pl.ds(sid * rps, rps)])

    return body


def _dense_stage(n_nodes, hidden, n_rel, n_cls, bn):
    def body(hp_ref, root1_ref, b1_ref, w2t_ref, root2_ref, b2_ref,
             p_ref, oroot_ref):
        h = hp_ref[0] + hp_ref[1] + root1_ref[...] + b1_ref[...]
        h = jnp.maximum(h, 0.0)
        oroot_ref[...] = (
            jnp.dot(h, root2_ref[...], preferred_element_type=jnp.float32)
            + b2_ref[...])
        p_ref[...] = jnp.dot(h, w2t_ref[...],
                             preferred_element_type=jnp.float32)

    return pl.pallas_call(
        body,
        grid=(n_nodes // bn,),
        in_specs=[
            pl.BlockSpec((2, bn, hidden), lambda i: (0, i, 0)),
            pl.BlockSpec((bn, hidden), lambda i: (i, 0)),
            pl.BlockSpec((1, hidden), lambda i: (0, 0)),
            pl.BlockSpec((hidden, n_rel * n_cls), lambda i: (0, 0)),
            pl.BlockSpec((hidden, n_cls), lambda i: (0, 0)),
            pl.BlockSpec((1, n_cls), lambda i: (0, 0)),
        ],
        out_specs=[
            pl.BlockSpec((bn, n_rel * n_cls), lambda i: (i, 0)),
            pl.BlockSpec((bn, n_cls), lambda i: (i, 0)),
        ],
        out_shape=[
            jax.ShapeDtypeStruct((n_nodes, n_rel * n_cls), jnp.float32),
            jax.ShapeDtypeStruct((n_nodes, n_cls), jnp.float32),
        ],
    )


def _final_stage(n_nodes, n_cls, bn):
    def body(o2_ref, oroot_ref, out_ref):
        x = o2_ref[0] + o2_ref[1] + oroot_ref[...]
        m = jnp.max(x, axis=1, keepdims=True)
        e = jnp.exp(x - m)
        lse = jnp.log(jnp.sum(e, axis=1, keepdims=True))
        out_ref[...] = x - m - lse

    return pl.pallas_call(
        body,
        grid=(n_nodes // bn,),
        in_specs=[
            pl.BlockSpec((2, bn, n_cls), lambda i: (0, i, 0)),
            pl.BlockSpec((bn, n_cls), lambda i: (i, 0)),
        ],
        out_specs=pl.BlockSpec((bn, n_cls), lambda i: (i, 0)),
        out_shape=jax.ShapeDtypeStruct((n_nodes, n_cls), jnp.float32),
    )


def kernel(edge_index, edge_type, W1, root1, b1, W2, root2, b2):
    n_rel, n_nodes, hidden = W1.shape
    n_cls = root2.shape[1]
    n_edges = edge_type.shape[0]

    chunk1, nbuf1 = 80, 1
    chunk2, nbuf2 = 80, 25
    epw = n_edges // NW

    src = edge_index[0]
    dst = edge_index[1]
    n_pad = ((n_nodes + 8 * NS - 1) // (8 * NS)) * (8 * NS)

    def _shard(x, fill, chunk, nbuf):
        n_chunks = -(-(-(-epw // chunk)) // nbuf) * nbuf
        pad = n_chunks * chunk - epw
        if fill is None:
            fill_vec = n_nodes + jnp.arange(pad, dtype=jnp.int32) % (
                n_pad - n_nodes)
            pad_blk = jnp.broadcast_to(fill_vec, (NW, pad))
            x = jnp.concatenate([x.reshape(NW, epw), pad_blk], axis=1)
        else:
            x = jnp.pad(x.reshape(NW, epw), ((0, 0), (0, pad)),
                        constant_values=fill)
        return x.reshape(NW, n_chunks, chunk), n_chunks

    idx1, nch1 = _shard(edge_type * n_nodes + src, 0, chunk1, nbuf1)
    idx2, nch2 = _shard(src * n_rel + edge_type, 0, chunk2, nbuf2)
    dstr1, _ = _shard(dst, None, chunk1, nbuf1)
    dstr2 = dstr1.reshape(NW, nch2, chunk2) if nch2 * chunk2 == nch1 * chunk1         else _shard(dst, None, chunk2, nbuf2)[0]
    zeros_h = jnp.zeros((n_pad, hidden), jnp.float32)
    zeros_c = jnp.zeros((n_pad, n_cls), jnp.float32)

    l1 = _sc_gather_scatter_add(n_pad, hidden, nch1, chunk1, nbuf1)
    hp = l1(W1.reshape(n_rel * n_nodes, hidden), idx1, dstr1, zeros_h)

    w2t = W2.transpose(1, 0, 2).reshape(hidden, n_rel * n_cls)
    p, oroot = _dense_stage(n_nodes, hidden, n_rel, n_cls, 2000)(
        hp, root1, b1.reshape(1, hidden), w2t, root2, b2.reshape(1, n_cls))

    l2 = _sc_gather_scatter_add(n_pad, n_cls, nch2, chunk2, nbuf2)
    out2 = l2(p.reshape(n_nodes * n_rel, n_cls), idx2, dstr2, zeros_c)

    return _final_stage(n_nodes, n_cls, 2000)(out2, oroot)
